# baseline (speedup 1.0000x reference)
import functools

import jax
import jax.numpy as jnp
from jax import lax
from jax.experimental import pallas as pl
from jax.experimental.pallas import tpu as pltpu
from jax.experimental.pallas import tpu_sc as plsc

B = 16
N = 512
K = 128
C = 128
CH = 16

_NC = 2
_NS = 16
_NW = _NC * _NS

_PAIRS = B * K
_PPW = _PAIRS // _NW
_SLAB = (N * CH) // 128
_EROWS_OUT = B * K * K * CH // 128


def _topk_body(sim_ref, nfidx_ref):
    b = pl.program_id(0)
    s = sim_ref[...].reshape(1, N)
    sj = jnp.broadcast_to(s, (N, N))
    si = sj.T
    ii = lax.broadcasted_iota(jnp.int32, (N, N), 0)
    jj = lax.broadcasted_iota(jnp.int32, (N, N), 1)
    prec = ((si < sj) | ((si == sj) & (ii < jj))).astype(jnp.float32)
    rank = jnp.sum(prec, axis=0, keepdims=True)
    sel = (rank < float(K)).astype(jnp.float32)
    tri = (ii < jj).astype(jnp.float32)
    pos = lax.dot_general(sel, tri, (((1,), (0,)), ((), ())),
                          preferred_element_type=jnp.float32)
    i128 = lax.broadcasted_iota(jnp.int32, (K, N), 0).astype(jnp.float32)
    onehot = ((jnp.broadcast_to(pos, (K, N)) == i128)
              & (jnp.broadcast_to(sel, (K, N)) > 0.0)).astype(jnp.float32)
    jr = lax.broadcasted_iota(jnp.int32, (1, N), 1)
    jhi = (jr // 2).astype(jnp.float32)
    jlo = (jr % 2).astype(jnp.float32)
    hi = lax.dot_general(jhi, onehot, (((1,), (1,)), ((), ())),
                         preferred_element_type=jnp.float32)
    lo = lax.dot_general(jlo, onehot, (((1,), (1,)), ((), ())),
                         preferred_element_type=jnp.float32)
    idx_row = 2.0 * hi + lo
    bf = b.astype(jnp.float32)
    nfidx_ref[...] = (bf * float(N) + idx_row).astype(jnp.int32).reshape(1, 1, K)


def _topk_indices(sim3):
    return pl.pallas_call(
        _topk_body,
        grid=(B,),
        in_specs=[pl.BlockSpec((1, 1, N), lambda b: (b, 0, 0))],
        out_specs=[pl.BlockSpec((1, 1, K), lambda b: (b, 0, 0))],
        out_shape=[jax.ShapeDtypeStruct((B, 1, K), jnp.int32)],
        compiler_params=pltpu.CompilerParams(
            dimension_semantics=("arbitrary",)),
    )(sim3)[0]


def _sc_gather_body(ntab, nidx, etab, nodes_out, e_out,
                    nidx_v, nbuf, jidx_v, slab_a, slab_b, ob_a, ob_b,
                    nsem, gsem_a, gsem_b, osem_a, osem_b):
    w = lax.axis_index("s") * _NC + lax.axis_index("c")
    b = w // 2
    base_pair = w * _PPW

    pltpu.sync_copy(nidx.at[pl.ds(base_pair, _PPW)], nidx_v.at[pl.ds(0, _PPW)])
    nidx_v[pl.ds(_PPW, 16)] = jnp.zeros((16,), jnp.int32)
    ncopy = pltpu.async_copy(ntab.at[nidx_v.at[pl.ds(0, _PPW)]], nbuf, nsem)
    ncopy.wait()
    pltpu.sync_copy(nbuf, nodes_out.at[pl.ds(base_pair, _PPW)])

    pltpu.sync_copy(nidx.at[pl.ds(b * K, K)], jidx_v)

    iota = lax.iota(jnp.int32, 16)
    pre = []
    for jb in range(8):
        jraw = jidx_v[pl.ds(jb * 16, 16)]
        jloc = jraw & (N - 1)
        row_in = jloc >> 3
        colb_in = (jloc & 7) << 4
        orow = (iota >> 3) + 2 * jb
        pre.append((row_in, colb_in, orow))
    ocol_b = (iota & 7) << 4

    def select(slab, ob):
        for jb in range(8):
            row_in, colb_in, orow = pre[jb]
            for ch in range(CH):
                v = plsc.load_gather(slab, [row_in, colb_in + ch])
                plsc.store_scatter(ob, [orow, ocol_b + ch], v)

    def slab_base(p):
        return nidx_v[pl.ds(p, 16)][0] * _SLAB

    def start_gather(p, slab, gsem):
        pltpu.async_copy(etab.at[pl.ds(slab_base(p), _SLAB)], slab, gsem)

    def wait_gather(slab, gsem):
        pltpu.make_async_copy(etab.at[pl.ds(0, _SLAB)], slab, gsem).wait()

    def out_row(p):
        return (base_pair + p) * (K * CH // 128)

    def phase(i, p_cur, slab, ob, gsem, osem, p_nxt, slab_nxt, gsem_nxt):
        wait_gather(slab, gsem)
        start_gather(p_nxt, slab_nxt, gsem_nxt)

        @pl.when(i > 0)
        def _():
            pltpu.make_async_copy(ob, e_out.at[pl.ds(0, K * CH // 128)],
                                  osem).wait()

        select(slab, ob)
        pltpu.async_copy(ob, e_out.at[pl.ds(out_row(p_cur), K * CH // 128)],
                         osem)

    start_gather(0, slab_a, gsem_a)

    def body(i, carry):
        p0 = 2 * i
        phase(i, p0, slab_a, ob_a, gsem_a, osem_a, p0 + 1, slab_b, gsem_b)
        phase(i, p0 + 1, slab_b, ob_b, gsem_b, osem_b, p0 + 2, slab_a, gsem_a)
        return carry

    lax.fori_loop(0, _PPW // 2, body, 0)

    wait_gather(slab_a, gsem_a)
    pltpu.make_async_copy(ob_a, e_out.at[pl.ds(0, K * CH // 128)], osem_a).wait()
    pltpu.make_async_copy(ob_b, e_out.at[pl.ds(0, K * CH // 128)], osem_b).wait()


@functools.lru_cache(maxsize=1)
def _sc_gather_fn():
    mesh = plsc.VectorSubcoreMesh(core_axis_name="c", subcore_axis_name="s",
                                  num_cores=_NC, num_subcores=_NS)
    return pl.kernel(
        _sc_gather_body,
        out_type=(jax.ShapeDtypeStruct((_PAIRS, C), jnp.float32),
                  jax.ShapeDtypeStruct((_EROWS_OUT, 128), jnp.float32)),
        mesh=mesh,
        scratch_types=[
            pltpu.VMEM((_PPW + 16,), jnp.int32),
            pltpu.VMEM((_PPW, C), jnp.float32),
            pltpu.VMEM((K,), jnp.int32),
            pltpu.VMEM((_SLAB, 128), jnp.float32),
            pltpu.VMEM((_SLAB, 128), jnp.float32),
            pltpu.VMEM((K * CH // 128, 128), jnp.float32),
            pltpu.VMEM((K * CH // 128, 128), jnp.float32),
            pltpu.SemaphoreType.DMA,
            pltpu.SemaphoreType.DMA,
            pltpu.SemaphoreType.DMA,
            pltpu.SemaphoreType.DMA,
            pltpu.SemaphoreType.DMA,
        ],
        compiler_params=pltpu.CompilerParams(needs_layout_passes=False),
    )


def kernel(obj_similarity, obj_mmt_in, obj_obj_edge_feat):
    nfidx = _topk_indices(obj_similarity.reshape(B, 1, N))
    nodes_flat, e_flat = _sc_gather_fn()(
        obj_mmt_in.reshape(B * N, C),
        nfidx.reshape(_PAIRS),
        obj_obj_edge_feat.reshape(B * N * N * CH // 128, 128),
    )
    nodes = nodes_flat.reshape(B, K, C)
    e = e_flat.reshape(B, K, K, CH)
    mask = jnp.ones((B, K), dtype=jnp.float32)
    return nodes, mask, e


# speedup vs baseline: 16.8870x; 16.8870x over previous
import functools

import jax
import jax.numpy as jnp
from jax import lax
from jax.experimental import pallas as pl
from jax.experimental.pallas import tpu as pltpu
from jax.experimental.pallas import tpu_sc as plsc

B = 16
N = 512
K = 128
C = 128
CH = 16

_NC = 2
_NS = 16
_NW = _NC * _NS

_PAIRS = B * K
_PPW = _PAIRS // _NW
_SLAB = (N * CH) // 128
_EROWS_OUT = B * K * K * CH // 128


def _topk_body(sim_ref, nfidx_ref):
    b = pl.program_id(0)
    s = sim_ref[...].reshape(1, N)
    sj = jnp.broadcast_to(s, (N, N))
    si = sj.T
    ii = lax.broadcasted_iota(jnp.int32, (N, N), 0)
    jj = lax.broadcasted_iota(jnp.int32, (N, N), 1)
    prec = ((si < sj) | ((si == sj) & (ii < jj))).astype(jnp.float32)
    rank = jnp.sum(prec, axis=0, keepdims=True)
    sel = (rank < float(K)).astype(jnp.float32)
    tri = (ii < jj).astype(jnp.float32)
    pos = lax.dot_general(sel, tri, (((1,), (0,)), ((), ())),
                          preferred_element_type=jnp.float32)
    i128 = lax.broadcasted_iota(jnp.int32, (K, N), 0).astype(jnp.float32)
    onehot = ((jnp.broadcast_to(pos, (K, N)) == i128)
              & (jnp.broadcast_to(sel, (K, N)) > 0.0)).astype(jnp.float32)
    jr = lax.broadcasted_iota(jnp.int32, (1, N), 1)
    jhi = (jr // 2).astype(jnp.float32)
    jlo = (jr % 2).astype(jnp.float32)
    hi = lax.dot_general(jhi, onehot, (((1,), (1,)), ((), ())),
                         preferred_element_type=jnp.float32)
    lo = lax.dot_general(jlo, onehot, (((1,), (1,)), ((), ())),
                         preferred_element_type=jnp.float32)
    idx_row = 2.0 * hi + lo
    bf = b.astype(jnp.float32)
    nfidx_ref[...] = (bf * float(N) + idx_row).astype(jnp.int32).reshape(1, 1, K)


def _topk_indices(sim3):
    return pl.pallas_call(
        _topk_body,
        grid=(B,),
        in_specs=[pl.BlockSpec((1, 1, N), lambda b: (b, 0, 0))],
        out_specs=[pl.BlockSpec((1, 1, K), lambda b: (b, 0, 0))],
        out_shape=[jax.ShapeDtypeStruct((B, 1, K), jnp.int32)],
        compiler_params=pltpu.CompilerParams(
            dimension_semantics=("arbitrary",)),
    )(sim3)[0]


def _sc_gather_body(ntab, nidx, etab, nodes_out, e_out,
                    nidx_v, nbuf, jidx_v, slab_a, slab_b, ob_a, ob_b,
                    nsem, gsem_a, gsem_b, osem_a, osem_b):
    w = lax.axis_index("s") * _NC + lax.axis_index("c")
    b = w // 2
    base_pair = w * _PPW

    pltpu.sync_copy(nidx.at[pl.ds(base_pair, _PPW)], nidx_v.at[pl.ds(0, _PPW)])
    nidx_v[pl.ds(_PPW, 16)] = jnp.zeros((16,), jnp.int32)
    ncopy = pltpu.async_copy(ntab.at[nidx_v.at[pl.ds(0, _PPW)]], nbuf, nsem)
    ncopy.wait()
    pltpu.sync_copy(nbuf, nodes_out.at[pl.ds(base_pair, _PPW)])

    pltpu.sync_copy(nidx.at[pl.ds(b * K, K)], jidx_v)

    pre = []
    for jb in range(8):
        jraw = jidx_v[pl.ds(jb * 16, 16)]
        jloc = jraw & (N - 1)
        jt8 = (jloc >> 7) << 3
        jl = jloc & 127
        pre.append((jt8, jl))

    def select(slab, ob):
        for jb in range(8):
            jt8, jl = pre[jb]
            for ch in range(CH):
                rowbase = (ch >> 3) * 32 + (ch & 7)
                v = plsc.load_gather(slab, [jt8 + rowbase, jl])
                ob[ch, pl.ds(jb * 16, 16)] = v

    def slab_base(p):
        return nidx_v[pl.ds(p, 16)][0] * _SLAB

    def start_gather(p, slab, gsem):
        pltpu.async_copy(etab.at[pl.ds(slab_base(p), _SLAB)], slab, gsem)

    def wait_gather(slab, gsem):
        pltpu.make_async_copy(etab.at[pl.ds(0, _SLAB)], slab, gsem).wait()

    def out_row(p):
        return (base_pair + p) * (K * CH // 128)

    def phase(i, p_cur, slab, ob, gsem, osem, p_nxt, slab_nxt, gsem_nxt):
        wait_gather(slab, gsem)
        start_gather(p_nxt, slab_nxt, gsem_nxt)

        @pl.when(i > 0)
        def _():
            pltpu.make_async_copy(ob, e_out.at[pl.ds(0, K * CH // 128)],
                                  osem).wait()

        select(slab, ob)
        pltpu.async_copy(ob, e_out.at[pl.ds(out_row(p_cur), K * CH // 128)],
                         osem)

    start_gather(0, slab_a, gsem_a)

    def body(i, carry):
        p0 = 2 * i
        phase(i, p0, slab_a, ob_a, gsem_a, osem_a, p0 + 1, slab_b, gsem_b)
        phase(i, p0 + 1, slab_b, ob_b, gsem_b, osem_b, p0 + 2, slab_a, gsem_a)
        return carry

    lax.fori_loop(0, _PPW // 2, body, 0)

    wait_gather(slab_a, gsem_a)
    pltpu.make_async_copy(ob_a, e_out.at[pl.ds(0, K * CH // 128)], osem_a).wait()
    pltpu.make_async_copy(ob_b, e_out.at[pl.ds(0, K * CH // 128)], osem_b).wait()


@functools.lru_cache(maxsize=1)
def _sc_gather_fn():
    mesh = plsc.VectorSubcoreMesh(core_axis_name="c", subcore_axis_name="s",
                                  num_cores=_NC, num_subcores=_NS)
    return pl.kernel(
        _sc_gather_body,
        out_type=(jax.ShapeDtypeStruct((_PAIRS, C), jnp.float32),
                  jax.ShapeDtypeStruct((_EROWS_OUT, 128), jnp.float32)),
        mesh=mesh,
        scratch_types=[
            pltpu.VMEM((_PPW + 16,), jnp.int32),
            pltpu.VMEM((_PPW, C), jnp.float32),
            pltpu.VMEM((K,), jnp.int32),
            pltpu.VMEM((_SLAB, 128), jnp.float32),
            pltpu.VMEM((_SLAB, 128), jnp.float32),
            pltpu.VMEM((K * CH // 128, 128), jnp.float32),
            pltpu.VMEM((K * CH // 128, 128), jnp.float32),
            pltpu.SemaphoreType.DMA,
            pltpu.SemaphoreType.DMA,
            pltpu.SemaphoreType.DMA,
            pltpu.SemaphoreType.DMA,
            pltpu.SemaphoreType.DMA,
        ],
        compiler_params=pltpu.CompilerParams(needs_layout_passes=False),
    )


def kernel(obj_similarity, obj_mmt_in, obj_obj_edge_feat):
    nfidx = _topk_indices(obj_similarity.reshape(B, 1, N))
    etab = (obj_obj_edge_feat
            .reshape(B, N, 4, 128, 2, 8)
            .transpose(0, 1, 4, 2, 5, 3)
            .reshape(B * N * N * CH // 128, 128))
    nodes_flat, e_flat = _sc_gather_fn()(
        obj_mmt_in.reshape(B * N, C),
        nfidx.reshape(_PAIRS),
        etab,
    )
    nodes = nodes_flat.reshape(B, K, C)
    e = (e_flat.reshape(B, K, 2, 8, K)
         .transpose(0, 1, 4, 2, 3)
         .reshape(B, K, K, CH))
    mask = jnp.ones((B, K), dtype=jnp.float32)
    return nodes, mask, e


# speedup vs baseline: 20.1257x; 1.1918x over previous
import functools

import jax
import jax.numpy as jnp
from jax import lax
from jax.experimental import pallas as pl
from jax.experimental.pallas import tpu as pltpu
from jax.experimental.pallas import tpu_sc as plsc

B = 16
N = 512
K = 128
C = 128
CH = 16

_NC = 2
_NS = 16
_NW = _NC * _NS

_PAIRS = B * K
_PPW = _PAIRS // _NW
_SLAB = (N * CH) // 128
_EROWS_OUT = B * K * K * CH // 128


def _topk_body(sim_ref, nfidx_ref):
    b = pl.program_id(0)
    s = sim_ref[...].reshape(1, N)
    sj = jnp.broadcast_to(s, (N, N))
    si = sj.T
    ii = lax.broadcasted_iota(jnp.int32, (N, N), 0)
    jj = lax.broadcasted_iota(jnp.int32, (N, N), 1)
    prec = ((si < sj) | ((si == sj) & (ii < jj))).astype(jnp.float32)
    rank = jnp.sum(prec, axis=0, keepdims=True)
    sel = (rank < float(K)).astype(jnp.float32)
    tri = (ii < jj).astype(jnp.float32)
    pos = lax.dot_general(sel, tri, (((1,), (0,)), ((), ())),
                          preferred_element_type=jnp.float32)
    i128 = lax.broadcasted_iota(jnp.int32, (K, N), 0).astype(jnp.float32)
    onehot = ((jnp.broadcast_to(pos, (K, N)) == i128)
              & (jnp.broadcast_to(sel, (K, N)) > 0.0)).astype(jnp.float32)
    jr = lax.broadcasted_iota(jnp.int32, (1, N), 1)
    jhi = (jr // 2).astype(jnp.float32)
    jlo = (jr % 2).astype(jnp.float32)
    hi = lax.dot_general(jhi, onehot, (((1,), (1,)), ((), ())),
                         preferred_element_type=jnp.float32)
    lo = lax.dot_general(jlo, onehot, (((1,), (1,)), ((), ())),
                         preferred_element_type=jnp.float32)
    idx_row = 2.0 * hi + lo
    bf = b.astype(jnp.float32)
    nfidx_ref[...] = (bf * float(N) + idx_row).astype(jnp.int32).reshape(1, 1, K)


def _topk_indices(sim3):
    return pl.pallas_call(
        _topk_body,
        grid=(B,),
        in_specs=[pl.BlockSpec((1, 1, N), lambda b: (b, 0, 0))],
        out_specs=[pl.BlockSpec((1, 1, K), lambda b: (b, 0, 0))],
        out_shape=[jax.ShapeDtypeStruct((B, 1, K), jnp.int32)],
        compiler_params=pltpu.CompilerParams(
            dimension_semantics=("arbitrary",)),
    )(sim3)[0]


_NBUF = 4
_SLABW = _SLAB * 128
_OBR = K * CH // 128


def _sc_gather_body(ntab, nidx, etab, nodes_out, e_out,
                    nidx_v, nbuf, jidx_v,
                    slab0, slab1, slab2, slab3, ob0, ob1, ob2, ob3,
                    nsem, gsem0, gsem1, gsem2, gsem3,
                    osem0, osem1, osem2, osem3):
    slabs = (slab0, slab1, slab2, slab3)
    obs = (ob0, ob1, ob2, ob3)
    gsems = (gsem0, gsem1, gsem2, gsem3)
    osems = (osem0, osem1, osem2, osem3)
    w = lax.axis_index("s") * _NC + lax.axis_index("c")
    b = w // 2
    base_pair = w * _PPW

    pltpu.sync_copy(nidx.at[pl.ds(base_pair, _PPW)], nidx_v.at[pl.ds(0, _PPW)])
    nidx_v[pl.ds(_PPW, 16)] = jnp.zeros((16,), jnp.int32)
    nidx_v[pl.ds(_PPW + 16, 16)] = jnp.zeros((16,), jnp.int32)
    ncopy = pltpu.async_copy(ntab.at[nidx_v.at[pl.ds(0, _PPW)]], nbuf, nsem)
    ncopy.wait()
    pltpu.sync_copy(nbuf, nodes_out.at[pl.ds(base_pair, _PPW)])

    pltpu.sync_copy(nidx.at[pl.ds(b * K, K)], jidx_v)

    pre = []
    for jb in range(8):
        jraw = jidx_v[pl.ds(jb * 16, 16)]
        jloc = jraw & (N - 1)
        fbase = ((jloc >> 7) << 10) + (jloc & 127)
        pre.append(fbase)

    def select(slab, ob):
        for jb in range(8):
            fbase = pre[jb]
            for ch in range(CH):
                off = (ch >> 3) * 4096 + (ch & 7) * 128
                v = plsc.load_gather(slab, [fbase + off])
                ob[ch, pl.ds(jb * 16, 16)] = v

    def start_gather(p, u):
        base = nidx_v[pl.ds(p, 16)][0] * _SLABW
        pltpu.async_copy(etab.at[pl.ds(base, _SLABW)], slabs[u], gsems[u])

    def wait_gather(u):
        pltpu.make_async_copy(etab.at[pl.ds(0, _SLABW)], slabs[u],
                              gsems[u]).wait()

    def wait_out(u):
        pltpu.make_async_copy(obs[u], e_out.at[pl.ds(0, _OBR)],
                              osems[u]).wait()

    for u in range(_NBUF - 1):
        start_gather(u, u)

    def body(i, carry):
        p0 = _NBUF * i
        for u in range(_NBUF):
            wait_gather(u)
            start_gather(p0 + u + (_NBUF - 1), (u + _NBUF - 1) % _NBUF)

            @pl.when(i > 0)
            def _():
                wait_out(u)

            select(slabs[u], obs[u])
            pltpu.async_copy(
                obs[u], e_out.at[pl.ds((base_pair + p0 + u) * _OBR, _OBR)],
                osems[u])
        return carry

    lax.fori_loop(0, _PPW // _NBUF, body, 0)

    for u in range(_NBUF - 1):
        wait_gather(u)
    for u in range(_NBUF):
        wait_out(u)


@functools.lru_cache(maxsize=1)
def _sc_gather_fn():
    mesh = plsc.VectorSubcoreMesh(core_axis_name="c", subcore_axis_name="s",
                                  num_cores=_NC, num_subcores=_NS)
    return pl.kernel(
        _sc_gather_body,
        out_type=(jax.ShapeDtypeStruct((_PAIRS, C), jnp.float32),
                  jax.ShapeDtypeStruct((_EROWS_OUT, 128), jnp.float32)),
        mesh=mesh,
        scratch_types=(
            [pltpu.VMEM((_PPW + 32,), jnp.int32),
             pltpu.VMEM((_PPW, C), jnp.float32),
             pltpu.VMEM((K,), jnp.int32)]
            + [pltpu.VMEM((_SLABW,), jnp.float32) for _ in range(_NBUF)]
            + [pltpu.VMEM((_OBR, 128), jnp.float32) for _ in range(_NBUF)]
            + [pltpu.SemaphoreType.DMA] * (1 + 2 * _NBUF)
        ),
        compiler_params=pltpu.CompilerParams(needs_layout_passes=False),
    )


def kernel(obj_similarity, obj_mmt_in, obj_obj_edge_feat):
    nfidx = _topk_indices(obj_similarity.reshape(B, 1, N))
    etab = (obj_obj_edge_feat
            .reshape(B, N, 4, 128, 2, 8)
            .transpose(0, 1, 4, 2, 5, 3)
            .reshape(B * N * N * CH))
    nodes_flat, e_flat = _sc_gather_fn()(
        obj_mmt_in.reshape(B * N, C),
        nfidx.reshape(_PAIRS),
        etab,
    )
    nodes = nodes_flat.reshape(B, K, C)
    e = (e_flat.reshape(B, K, 2, 8, K)
         .transpose(0, 1, 4, 2, 3)
         .reshape(B, K, K, CH))
    mask = jnp.ones((B, K), dtype=jnp.float32)
    return nodes, mask, e


# speedup vs baseline: 24.1159x; 1.1983x over previous
import functools

import jax
import jax.numpy as jnp
from jax import lax
from jax.experimental import pallas as pl
from jax.experimental.pallas import tpu as pltpu
from jax.experimental.pallas import tpu_sc as plsc

B = 16
N = 512
K = 128
C = 128
CH = 16

_NC = 2
_NS = 16
_NW = _NC * _NS

_PAIRS = B * K
_PPW = _PAIRS // _NW
_SLAB = (N * CH) // 128
_EROWS_OUT = B * K * K * CH // 128


def _topk_body(sim_ref, nfidx_ref):
    b = pl.program_id(0)
    s = sim_ref[...].reshape(1, N)
    sj = jnp.broadcast_to(s, (N, N))
    si = sj.T
    ii = lax.broadcasted_iota(jnp.int32, (N, N), 0)
    jj = lax.broadcasted_iota(jnp.int32, (N, N), 1)
    prec = ((si < sj) | ((si == sj) & (ii < jj))).astype(jnp.float32)
    rank = jnp.sum(prec, axis=0, keepdims=True)
    sel = (rank < float(K)).astype(jnp.float32)
    tri = (ii < jj).astype(jnp.float32)
    pos = lax.dot_general(sel, tri, (((1,), (0,)), ((), ())),
                          preferred_element_type=jnp.float32)
    i128 = lax.broadcasted_iota(jnp.int32, (K, N), 0).astype(jnp.float32)
    onehot = ((jnp.broadcast_to(pos, (K, N)) == i128)
              & (jnp.broadcast_to(sel, (K, N)) > 0.0)).astype(jnp.float32)
    jr = lax.broadcasted_iota(jnp.int32, (1, N), 1)
    jhi = (jr // 2).astype(jnp.float32)
    jlo = (jr % 2).astype(jnp.float32)
    hi = lax.dot_general(jhi, onehot, (((1,), (1,)), ((), ())),
                         preferred_element_type=jnp.float32)
    lo = lax.dot_general(jlo, onehot, (((1,), (1,)), ((), ())),
                         preferred_element_type=jnp.float32)
    idx_row = 2.0 * hi + lo
    bf = b.astype(jnp.float32)
    nfidx_ref[...] = (bf * float(N) + idx_row).astype(jnp.int32).reshape(1, 1, K)


def _topk_indices(sim3):
    return pl.pallas_call(
        _topk_body,
        grid=(B,),
        in_specs=[pl.BlockSpec((1, 1, N), lambda b: (b, 0, 0))],
        out_specs=[pl.BlockSpec((1, 1, K), lambda b: (b, 0, 0))],
        out_shape=[jax.ShapeDtypeStruct((B, 1, K), jnp.int32)],
        compiler_params=pltpu.CompilerParams(
            dimension_semantics=("arbitrary",)),
    )(sim3)[0]


_NBUF = 4
_SLABW = _SLAB * 128
_OBR = K * CH // 128


def _sc_gather_body(ntab, nidx, etab, nodes_out, e_out,
                    nidx_v, nbuf, jidx_v,
                    slab0, slab1, slab2, slab3, ob0, ob1, ob2, ob3,
                    nsem, gsem0, gsem1, gsem2, gsem3,
                    osem0, osem1, osem2, osem3):
    slabs = (slab0, slab1, slab2, slab3)
    obs = (ob0, ob1, ob2, ob3)
    gsems = (gsem0, gsem1, gsem2, gsem3)
    osems = (osem0, osem1, osem2, osem3)
    w = lax.axis_index("s") * _NC + lax.axis_index("c")
    b = w // 2
    base_pair = w * _PPW

    pltpu.sync_copy(nidx.at[pl.ds(base_pair, _PPW)], nidx_v.at[pl.ds(0, _PPW)])
    nidx_v[pl.ds(_PPW, 16)] = jnp.zeros((16,), jnp.int32)
    nidx_v[pl.ds(_PPW + 16, 16)] = jnp.zeros((16,), jnp.int32)
    ncopy = pltpu.async_copy(ntab.at[nidx_v.at[pl.ds(0, _PPW)]], nbuf, nsem)
    ncopy.wait()
    pltpu.sync_copy(nbuf, nodes_out.at[pl.ds(base_pair, _PPW)])

    pltpu.sync_copy(nidx.at[pl.ds(b * K, K)], jidx_v)

    pre = []
    for jb in range(8):
        jraw = jidx_v[pl.ds(jb * 16, 16)]
        jloc = jraw & (N - 1)
        fbase = ((jloc >> 7) << 10) + (jloc & 127)
        pre.append(fbase)

    _ORDER = [(jb, ch) for jb in range(8) for ch in range(CH)]
    _D = 6

    def select(slab, ob):
        vals = {}
        for t in range(len(_ORDER) + _D):
            if t < len(_ORDER):
                jb, ch = _ORDER[t]
                off = (ch >> 3) * 4096 + (ch & 7) * 128
                vals[t] = plsc.load_gather(slab, [pre[jb] + off])
            if t >= _D:
                jb, ch = _ORDER[t - _D]
                ob[ch, pl.ds(jb * 16, 16)] = vals.pop(t - _D)

    def start_gather(p, u):
        base = nidx_v[pl.ds(p, 16)][0] * _SLABW
        pltpu.async_copy(etab.at[pl.ds(base, _SLABW)], slabs[u], gsems[u])

    def wait_gather(u):
        pltpu.make_async_copy(etab.at[pl.ds(0, _SLABW)], slabs[u],
                              gsems[u]).wait()

    def wait_out(u):
        pltpu.make_async_copy(obs[u], e_out.at[pl.ds(0, _OBR)],
                              osems[u]).wait()

    for u in range(_NBUF - 1):
        start_gather(u, u)

    def body(i, carry):
        p0 = _NBUF * i
        for u in range(_NBUF):
            wait_gather(u)
            start_gather(p0 + u + (_NBUF - 1), (u + _NBUF - 1) % _NBUF)

            @pl.when(i > 0)
            def _():
                wait_out(u)

            select(slabs[u], obs[u])
            pltpu.async_copy(
                obs[u], e_out.at[pl.ds((base_pair + p0 + u) * _OBR, _OBR)],
                osems[u])
        return carry

    lax.fori_loop(0, _PPW // _NBUF, body, 0)

    for u in range(_NBUF - 1):
        wait_gather(u)
    for u in range(_NBUF):
        wait_out(u)


@functools.lru_cache(maxsize=1)
def _sc_gather_fn():
    mesh = plsc.VectorSubcoreMesh(core_axis_name="c", subcore_axis_name="s",
                                  num_cores=_NC, num_subcores=_NS)
    return pl.kernel(
        _sc_gather_body,
        out_type=(jax.ShapeDtypeStruct((_PAIRS, C), jnp.float32),
                  jax.ShapeDtypeStruct((_EROWS_OUT, 128), jnp.float32)),
        mesh=mesh,
        scratch_types=(
            [pltpu.VMEM((_PPW + 32,), jnp.int32),
             pltpu.VMEM((_PPW, C), jnp.float32),
             pltpu.VMEM((K,), jnp.int32)]
            + [pltpu.VMEM((_SLABW,), jnp.float32) for _ in range(_NBUF)]
            + [pltpu.VMEM((_OBR, 128), jnp.float32) for _ in range(_NBUF)]
            + [pltpu.SemaphoreType.DMA] * (1 + 2 * _NBUF)
        ),
        compiler_params=pltpu.CompilerParams(needs_layout_passes=False),
    )


def kernel(obj_similarity, obj_mmt_in, obj_obj_edge_feat):
    nfidx = _topk_indices(obj_similarity.reshape(B, 1, N))
    etab = (obj_obj_edge_feat
            .reshape(B, N, 4, 128, 2, 8)
            .transpose(0, 1, 4, 2, 5, 3)
            .reshape(B * N * N * CH))
    nodes_flat, e_flat = _sc_gather_fn()(
        obj_mmt_in.reshape(B * N, C),
        nfidx.reshape(_PAIRS),
        etab,
    )
    nodes = nodes_flat.reshape(B, K, C)
    e = (e_flat.reshape(B, K, 2, 8, K)
         .transpose(0, 1, 4, 2, 3)
         .reshape(B, K, K, CH))
    mask = jnp.ones((B, K), dtype=jnp.float32)
    return nodes, mask, e
